# Initial kernel scaffold; baseline (speedup 1.0000x reference)
import jax
import jax.numpy as jnp
from jax.experimental import pallas as pl


def kernel(token_input, segment_input, token_table, segment_table, pos_table, gamma, beta):
    raise NotImplementedError("write your pallas kernel here")


# speedup vs baseline: 1.0409x; 1.0409x over previous
import functools

import jax
import jax.numpy as jnp
from jax import lax
from jax.experimental import pallas as pl
from jax.experimental.pallas import tpu as pltpu
from jax.experimental.pallas import tpu_sc as plsc

B = 1024
S = 200
D = 64
N = B * S
NW = 32
PER_W = N // NW
C = 640
NCHUNK = PER_W // C
CB = C // 128
U = 16
EPS = 1e-6


def _rsqrt_vec(x):
    i = lax.bitcast_convert_type(x, jnp.int32)
    i = jnp.full((16,), jnp.int32(0x5F3759DF)) - lax.shift_right_logical(i, 1)
    y = lax.bitcast_convert_type(i, jnp.float32)
    h = x * 0.5
    for _ in range(3):
        y = y * (1.5 - h * y * y)
    return y


def _body(tok_idx, seg_idx, table, pos, segtab, gamma, beta, out,
          tok_idx_v, seg_idx_v, rows_v, pos_v, segtab_v, gam_v, bet_v, sem):
    wid = lax.axis_index("s") * 2 + lax.axis_index("c")

    pltpu.sync_copy(pos, pos_v)
    pltpu.sync_copy(segtab, segtab_v)
    pltpu.sync_copy(gamma, gam_v)
    pltpu.sync_copy(beta, bet_v)

    s0 = [segtab_v[0, pl.ds(16 * j, 16)] for j in range(4)]
    s1 = [segtab_v[1, pl.ds(16 * j, 16)] for j in range(4)]
    sd = [s1[j] - s0[j] for j in range(4)]
    gv = [gam_v[pl.ds(16 * j, 16)] for j in range(4)]
    bv = [bet_v[pl.ds(16 * j, 16)] for j in range(4)]
    lane = lax.iota(jnp.int32, 16)
    dnums = lax.GatherDimensionNumbers(
        offset_dims=(), collapsed_slice_dims=(0,), start_index_map=(0,))
    perms = [
        lax.bitwise_xor(lane, jnp.full((16,), jnp.int32(m))).reshape(16, 1)
        for m in (8, 4, 2, 1)
    ]

    def allsum(v):
        for perm in perms:
            v = v + lax.gather(
                v, perm, dnums, (1,),
                mode=lax.GatherScatterMode.PROMISE_IN_BOUNDS)
        return v

    def chunk_body(k, _):
        base = wid * PER_W + k * C
        base = pl.multiple_of(base, 128)
        pltpu.sync_copy(tok_idx.at[pl.ds(base, C)], tok_idx_v)
        seg_off = pl.multiple_of(base // U, 8)
        pltpu.sync_copy(seg_idx.at[pl.ds(seg_off, C // U)], seg_idx_v)

        cps = [
            pltpu.async_copy(
                table.at[tok_idx_v.at[pl.ds(j * 128, 128)]],
                rows_v.at[pl.ds(j * 128, 128)],
                sem,
            )
            for j in range(CB)
        ]
        for cp in cps:
            cp.wait()

        def row_body(i, _):
            segf = seg_idx_v[i].astype(jnp.float32)
            for u in range(U):
                r = i * U + u
                p = (base + r) % S
                tb = jnp.full((16,), segf[u])
                x = [
                    rows_v[r, pl.ds(16 * j, 16)]
                    + pos_v[p, pl.ds(16 * j, 16)]
                    + (s0[j] + tb * sd[j])
                    for j in range(4)
                ]
                ssum = allsum((x[0] + x[1]) + (x[2] + x[3]))
                qsum = allsum(
                    (x[0] * x[0] + x[1] * x[1])
                    + (x[2] * x[2] + x[3] * x[3])
                )
                mb = ssum * (1.0 / D)
                var = qsum * (1.0 / D) - mb * mb
                rb = _rsqrt_vec(var + EPS)
                for j in range(4):
                    o = (x[j] - mb) * rb * gv[j] + bv[j]
                    rows_v[r, pl.ds(16 * j, 16)] = o
            return ()

        lax.fori_loop(0, C // U, row_body, (), unroll=False)
        pltpu.sync_copy(rows_v, out.at[pl.ds(base, C)])
        return ()

    lax.fori_loop(0, NCHUNK, chunk_body, (), unroll=False)


def kernel(token_input, segment_input, token_table, segment_table, pos_table,
           gamma, beta):
    tok2d = token_input.reshape(N)
    seg2d = segment_input.reshape(N // U, U)

    mesh = plsc.VectorSubcoreMesh(core_axis_name="c", subcore_axis_name="s")
    run = functools.partial(
        pl.kernel,
        mesh=mesh,
        compiler_params=pltpu.CompilerParams(use_tc_tiling_on_sc=False),
        out_type=jax.ShapeDtypeStruct((N, D), jnp.float32),
        scratch_types=[
            pltpu.VMEM((C,), jnp.int32),
            pltpu.VMEM((C // U, U), jnp.int32),
            pltpu.VMEM((C, D), jnp.float32),
            pltpu.VMEM((S, D), jnp.float32),
            pltpu.VMEM((2, D), jnp.float32),
            pltpu.VMEM((D,), jnp.float32),
            pltpu.VMEM((D,), jnp.float32),
            pltpu.SemaphoreType.DMA,
        ],
    )(_body)
    out = run(tok2d, seg2d, token_table, pos_table, segment_table, gamma, beta)
    return out.reshape(B, S, D)
